# Initial kernel scaffold; baseline (speedup 1.0000x reference)
import jax
import jax.numpy as jnp
from jax.experimental import pallas as pl


def kernel(binSets, embeds, THRESHOLD):
    raise NotImplementedError("write your pallas kernel here")


# speedup vs baseline: 33.8791x; 33.8791x over previous
import jax
import jax.numpy as jnp
from jax import lax
from jax.experimental import pallas as pl
from jax.experimental.pallas import tpu as pltpu

_S = 8
_L = 32
_M = 8
_B = _S * _L
_D = 128
_SENT = jnp.iinfo(jnp.int32).max


def _match_body(bins_ref, thr_ref, emb_ref, ba_ref, mv_ref,
                table_ref, glob_ref, loc_ref, cnt_ref):
    i32 = jnp.int32
    f32 = jnp.float32
    inf = jnp.float32(jnp.inf)

    def _gather(i, carry):
        s = i // _L
        c = i - s * _L
        idx = bins_ref[s, c]
        table_ref[pl.ds(i, 1), :] = emb_ref[pl.ds(idx, 1), :]
        return carry

    lax.fori_loop(0, _B, _gather, 0)

    bi = lax.broadcasted_iota(i32, (_B, _M), 0)
    ji = lax.broadcasted_iota(i32, (_B, _M), 1)
    glob_ref[...] = jnp.full((_B, _M), _SENT, i32)
    loc_ref[...] = jnp.where((ji == 0) & (bi < _L), bi, 0)
    bcol = lax.broadcasted_iota(i32, (_B, 1), 0)
    cnt_ref[...] = jnp.where(bcol < _L, 1, 0).astype(i32)

    def _init0(i, carry):
        v = bins_ref[0, i]
        glob_ref[pl.ds(i, 1), pl.ds(0, 1)] = v.reshape(1, 1)
        return carry

    lax.fori_loop(0, _L, _init0, 0)

    tab = table_ref[...]
    tabT = jnp.swapaxes(tab, 0, 1)

    def _comp(t):
        acc = None
        for i in range(_D // 8):
            f = 8 * i + t
            d = tab[:, f:f + 1] - tabT[f:f + 1, :]
            sq = d * d
            acc = sq if acc is None else acc + sq
        return acc

    s01 = _comp(0) + _comp(1)
    s23 = _comp(2) + _comp(3)
    s45 = _comp(4) + _comp(5)
    s67 = _comp(6) + _comp(7)
    dfull = jnp.sqrt((s01 + s23) + (s45 + s67))

    lane32 = lax.broadcasted_iota(i32, (1, _L), 1)
    lane8 = lax.broadcasted_iota(i32, (1, _M), 1)
    kiota = lax.broadcasted_iota(i32, (_B, _B), 1)
    biota = lax.broadcasted_iota(i32, (_B, _L), 0)
    flatio = biota * _L + lax.broadcasted_iota(i32, (_B, _L), 1)
    thr = thr_ref[0, 0]

    ba_ref[pl.ds(0, 1), :] = lane32
    nbins = jnp.int32(_L)

    for cand in range(1, _S):
        dcols = dfull[:, _L * cand:_L * (cand + 1)]
        locv = loc_ref[...]
        cntv = cnt_ref[...]

        dms = []
        for j in range(_M):
            oh = (locv[:, j:j + 1] == kiota).astype(f32)
            dj = lax.dot_general(oh, dcols, (((1,), (0,)), ((), ())),
                                 precision=lax.Precision.HIGHEST,
                                 preferred_element_type=f32)
            dms.append(jnp.where(cntv > j, dj, jnp.float32(0.0)))
        seq = dms[0]
        for j in range(1, _M):
            seq = seq + dms[j]
        tree = ((dms[0] + dms[1]) + (dms[2] + dms[3])) + (
            (dms[4] + dms[5]) + (dms[6] + dms[7]))
        ssum = jnp.where(cntv == _M, tree, seq)
        mean = ssum / jnp.maximum(cntv, 1).astype(f32)
        mat = jnp.where(biota < nbins, mean, inf)

        def _greedy(k, carry):
            rowpen, colpen, rsv, csv, vsv = carry
            masked = (mat + rowpen) + colpen
            m = jnp.min(masked)
            idx = jnp.min(jnp.where(masked == m, flatio, jnp.int32(2 ** 30)))
            r = idx // _L
            c = idx - r * _L
            rsv = jnp.where(lane32 == k, r, rsv)
            csv = jnp.where(lane32 == k, c, csv)
            vsv = jnp.where(lane32 == k, m, vsv)
            rowpen = jnp.where(bcol == r, inf, rowpen)
            colpen = jnp.where(lane32 == c, inf, colpen)
            return rowpen, colpen, rsv, csv, vsv

        carry0 = (jnp.zeros((_B, 1), f32), jnp.zeros((1, _L), f32),
                  jnp.zeros((1, _L), i32), jnp.zeros((1, _L), i32),
                  jnp.zeros((1, _L), f32))
        _, _, rsv, csv, vsv = lax.fori_loop(0, _L, _greedy, carry0)

        mv_ref[pl.ds(cand - 1, 1), :] = vsv

        a = vsv[:, 1:]
        nb = -vsv[:, :_L - 1]
        ssm = a + nb
        bv = ssm - a
        av = ssm - bv
        e = (a - av) + (nb - bv)
        exceeds = (ssm > thr) | ((ssm == thr) & (e > jnp.float32(0.0)))
        flagi = jnp.concatenate(
            [jnp.ones((1, 1), i32), 1 - exceeds.astype(i32)], axis=1)

        def _update(k, carry):
            nbins_c, ba_row = carry
            onek = (lane32 == k)
            r = jnp.sum(jnp.where(onek, rsv, 0))
            c = jnp.sum(jnp.where(onek, csv, 0))
            f = jnp.sum(jnp.where(onek, flagi, 0)) > 0
            tb = jnp.where(f, r, nbins_c)
            v = bins_ref[cand, c]
            rowg = glob_ref[pl.ds(tb, 1), :]
            rowl = loc_ref[pl.ds(tb, 1), :]
            present = jnp.sum((rowg == v).astype(i32)) > 0
            p = jnp.sum((rowg < v).astype(i32))
            shg = jnp.concatenate([rowg[:, :1], rowg[:, :_M - 1]], axis=1)
            shl = jnp.concatenate([rowl[:, :1], rowl[:, :_M - 1]], axis=1)
            newg = jnp.where(lane8 < p, rowg, jnp.where(lane8 == p, v, shg))
            newloc = jnp.int32(cand * _L) + c
            newl = jnp.where(lane8 < p, rowl,
                             jnp.where(lane8 == p, newloc, shl))
            glob_ref[pl.ds(tb, 1), :] = jnp.where(present, rowg, newg)
            loc_ref[pl.ds(tb, 1), :] = jnp.where(present, rowl, newl)
            inc = jnp.where(present, 0, 1).astype(i32)
            cnt_ref[...] = cnt_ref[...] + jnp.where(bcol == tb, inc, 0)
            ba_row = jnp.where(lane32 == c, tb, ba_row)
            nbins_c = nbins_c + jnp.where(f, 0, 1).astype(i32)
            return nbins_c, ba_row

        nbins, ba_row = lax.fori_loop(
            0, _L, _update, (nbins, jnp.full((1, _L), -1, i32)))
        ba_ref[pl.ds(cand, 1), :] = ba_row


def kernel(binSets, embeds, THRESHOLD):
    bins = jnp.asarray(binSets, jnp.int32)
    emb = jnp.asarray(embeds, jnp.float32)
    thr = jnp.asarray(THRESHOLD).astype(jnp.float32).reshape(1, 1)
    ba, mv = pl.pallas_call(
        _match_body,
        in_specs=[
            pl.BlockSpec(memory_space=pltpu.SMEM),
            pl.BlockSpec(memory_space=pltpu.SMEM),
            pl.BlockSpec(memory_space=pltpu.VMEM),
        ],
        out_specs=[
            pl.BlockSpec(memory_space=pltpu.VMEM),
            pl.BlockSpec(memory_space=pltpu.VMEM),
        ],
        out_shape=[
            jax.ShapeDtypeStruct((_S, _L), jnp.int32),
            jax.ShapeDtypeStruct((_S - 1, _L), jnp.float32),
        ],
        scratch_shapes=[
            pltpu.VMEM((_B, _D), jnp.float32),
            pltpu.VMEM((_B, _M), jnp.int32),
            pltpu.VMEM((_B, _M), jnp.int32),
            pltpu.VMEM((_B, 1), jnp.int32),
        ],
    )(bins, thr, emb)
    return ba, jnp.reshape(mv, (-1,))
